# Initial kernel scaffold; baseline (speedup 1.0000x reference)
import functools

import jax
import jax.numpy as jnp
from jax import lax
from jax.experimental import pallas as pl
from jax.experimental.pallas import tpu as pltpu
from jax.experimental.pallas import tpu_sc as plsc

BATCH = 16384
FIELDS = 26
EMBED_DIM = 16

NUM_CORES = 2
NUM_SUBCORES = 16
NUM_WORKERS = NUM_CORES * NUM_SUBCORES
TOTAL = BATCH * FIELDS
PER_WORKER = TOTAL // NUM_WORKERS
CHUNK = 1664
NUM_CHUNKS = PER_WORKER // CHUNK

_mesh = plsc.VectorSubcoreMesh(
    core_axis_name="c", subcore_axis_name="s",
    num_cores=NUM_CORES, num_subcores=NUM_SUBCORES)


@functools.partial(
    pl.kernel,
    out_type=jax.ShapeDtypeStruct((TOTAL, 2 * EMBED_DIM), jnp.float32),
    mesh=_mesh,
    scratch_types=[
        pltpu.VMEM((CHUNK,), jnp.int32),
        pltpu.VMEM((CHUNK, EMBED_DIM), jnp.float32),
        pltpu.VMEM((CHUNK, EMBED_DIM), jnp.float32),
        pltpu.SemaphoreType.DMA,
        pltpu.SemaphoreType.DMA,
    ],
)
def _stack_embed(x_hbm, t0_hbm, t1_hbm, out_hbm, idx_v, r0_v, r1_v, s0, s1):
    wid = lax.axis_index("s") * NUM_CORES + lax.axis_index("c")
    base = wid * PER_WORKER

    def body(i, carry):
        off = pl.multiple_of(base + i * CHUNK, 8)
        pltpu.sync_copy(x_hbm.at[pl.ds(off, CHUNK)], idx_v)
        cp0 = pltpu.async_copy(t0_hbm.at[idx_v], r0_v, s0)
        cp1 = pltpu.async_copy(t1_hbm.at[idx_v], r1_v, s1)
        cp0.wait()
        cp1.wait()
        pltpu.sync_copy(r0_v, out_hbm.at[pl.ds(off, CHUNK), pl.ds(0, EMBED_DIM)])
        pltpu.sync_copy(r1_v, out_hbm.at[pl.ds(off, CHUNK), pl.ds(EMBED_DIM, EMBED_DIM)])
        return carry

    lax.fori_loop(0, NUM_CHUNKS, body, 0)


def kernel(x, table0, table1):
    xf = x.reshape(TOTAL).astype(jnp.int32)
    out = _stack_embed(xf, table0, table1)
    return out.reshape(BATCH, FIELDS, 2 * EMBED_DIM)


# speedup vs baseline: 1.0861x; 1.0861x over previous
import functools

import jax
import jax.numpy as jnp
from jax import lax
from jax.experimental import pallas as pl
from jax.experimental.pallas import tpu as pltpu
from jax.experimental.pallas import tpu_sc as plsc

BATCH = 16384
FIELDS = 26
EMBED_DIM = 16

NUM_CORES = 2
NUM_SUBCORES = 16
NUM_WORKERS = NUM_CORES * NUM_SUBCORES
TOTAL = BATCH * FIELDS
PER_WORKER = TOTAL // NUM_WORKERS
CHUNK = 1664
NUM_CHUNKS = PER_WORKER // CHUNK

_mesh = plsc.VectorSubcoreMesh(
    core_axis_name="c", subcore_axis_name="s",
    num_cores=NUM_CORES, num_subcores=NUM_SUBCORES)


@functools.partial(
    pl.kernel,
    out_type=jax.ShapeDtypeStruct((TOTAL, 2 * EMBED_DIM), jnp.float32),
    mesh=_mesh,
    scratch_types=[
        pltpu.VMEM((CHUNK,), jnp.int32),
        pltpu.VMEM((CHUNK, EMBED_DIM), jnp.float32),
        pltpu.VMEM((CHUNK, EMBED_DIM), jnp.float32),
        pltpu.SemaphoreType.DMA,
        pltpu.SemaphoreType.DMA,
    ],
    compiler_params=pltpu.CompilerParams(use_tc_tiling_on_sc=False),
)
def _stack_embed(x_hbm, t0_hbm, t1_hbm, out_hbm, idx_v, r0_v, r1_v, s0, s1):
    wid = lax.axis_index("s") * NUM_CORES + lax.axis_index("c")
    base = wid * PER_WORKER

    def body(i, carry):
        off = pl.multiple_of(base + i * CHUNK, 8)
        pltpu.sync_copy(x_hbm.at[pl.ds(off, CHUNK)], idx_v)
        cp0 = pltpu.async_copy(t0_hbm.at[idx_v], r0_v, s0)
        cp1 = pltpu.async_copy(t1_hbm.at[idx_v], r1_v, s1)
        cp0.wait()
        cp1.wait()
        pltpu.sync_copy(r0_v, out_hbm.at[pl.ds(off, CHUNK), pl.ds(0, EMBED_DIM)])
        pltpu.sync_copy(r1_v, out_hbm.at[pl.ds(off, CHUNK), pl.ds(EMBED_DIM, EMBED_DIM)])
        return carry

    lax.fori_loop(0, NUM_CHUNKS, body, 0)


def kernel(x, table0, table1):
    xf = x.reshape(TOTAL).astype(jnp.int32)
    out = _stack_embed(xf, table0, table1)
    return out.reshape(BATCH, FIELDS, 2 * EMBED_DIM)
